# Initial kernel scaffold; baseline (speedup 1.0000x reference)
import jax
import jax.numpy as jnp
from jax.experimental import pallas as pl


def kernel(state, x, edge_index, edge_weight, emb_table, gcn_W, gcn_b, fc1_W, fc1_b, fc2_W, fc2_b, fc3_W, fc3_b, val_W, val_b, adv_W, adv_b):
    raise NotImplementedError("write your pallas kernel here")


# speedup vs baseline: 15.3439x; 15.3439x over previous
import functools

import jax
import jax.numpy as jnp
from jax import lax
from jax.experimental import pallas as pl
from jax.experimental.pallas import tpu as pltpu
from jax.experimental.pallas import tpu_sc as plsc


def _rsqrt_newton(d):
    i = plsc.bitcast(d, jnp.int32)
    i = jnp.int32(0x5F3759DF) - lax.shift_right_logical(i, 1)
    y = plsc.bitcast(i, jnp.float32)
    for _ in range(3):
        y = y * (1.5 - 0.5 * d * y * y)
    return y


def _sc_coeffs(N, E, S, B, EMB):
    L = 16
    NT = 16
    NL = NT * L
    N2 = ((N + NL - 1) // NL) * NL
    S2 = ((S + NL - 1) // NL) * NL
    npt = N2 // NT
    spt = S2 // NT
    EC = 128
    assert E % EC == 0
    n_ech = E // EC
    ech_pt = -(-n_ech // NT)
    CC = 80
    assert N % CC == 0
    n_cch = N // CC
    cch_pt = -(-n_cch // NT)
    bpt = B // NT
    assert B % NT == 0

    mesh = plsc.VectorSubcoreMesh(core_axis_name="c", subcore_axis_name="s")

    @functools.partial(
        pl.kernel,
        out_type=[
            jax.ShapeDtypeStruct((S2,), jnp.float32),
            jax.ShapeDtypeStruct((B, EMB), jnp.float32),
        ],
        mesh=mesh,
        compiler_params=pltpu.CompilerParams(needs_layout_passes=False,
                                             use_tc_tiling_on_sc=False),
        scratch_types=[
            pltpu.VMEM_SHARED((N2,), jnp.float32),
            pltpu.VMEM_SHARED((N2,), jnp.float32),
            pltpu.VMEM_SHARED((S2,), jnp.float32),
            pltpu.VMEM((spt,), jnp.float32),
            pltpu.VMEM((npt,), jnp.float32),
            pltpu.VMEM((EC,), jnp.int32),
            pltpu.VMEM((EC,), jnp.int32),
            pltpu.VMEM((EC,), jnp.float32),
            pltpu.VMEM((EC,), jnp.float32),
            pltpu.VMEM((CC,), jnp.int32),
            pltpu.VMEM((CC,), jnp.float32),
            pltpu.VMEM((bpt,), jnp.int32),
            pltpu.VMEM((bpt, EMB), jnp.float32),
        ],
    )
    def sc_kernel(row_h, col_h, w_h, x_h, st_h, emb_h, cc_h, x1_h,
                  deg_sh, u_sh, cc_sh, wb, nbuf, eci, eri, ew, eg, cxi, ccv,
                  sti, srows):
        cid = lax.axis_index("c")
        sid = lax.axis_index("s")

        @pl.when(cid == 0)
        def _core0():
            t = sid
            nb = t * npt

            def zero_loop(j, carry):
                wb[pl.ds(j * L, L)] = jnp.zeros((L,), jnp.float32)
                return carry
            lax.fori_loop(0, spt // L, zero_loop, None)
            pltpu.sync_copy(wb.at[pl.ds(0, npt)], deg_sh.at[pl.ds(nb, npt)])
            pltpu.sync_copy(wb.at[pl.ds(0, npt)], u_sh.at[pl.ds(nb, npt)])
            pltpu.sync_copy(wb, cc_sh.at[pl.ds(t * spt, spt)])
            plsc.subcore_barrier()

            lo = t * ech_pt
            hi = jnp.minimum(lo + ech_pt, n_ech)

            def pass_a(i, carry):
                b = pl.multiple_of(i * EC, EC)
                pltpu.sync_copy(col_h.at[pl.ds(b, EC)], eci)
                pltpu.sync_copy(w_h.at[pl.ds(b, EC)], ew)
                pltpu.sync_copy(ew, deg_sh.at[eci], add=True)
                return carry
            lax.fori_loop(lo, hi, pass_a, None)
            plsc.subcore_barrier()

            pltpu.sync_copy(deg_sh.at[pl.ds(nb, npt)], nbuf)

            def dis_loop(j, carry):
                sl = pl.ds(j * L, L)
                nbuf[sl] = _rsqrt_newton(nbuf[sl] + 1.0)
                return carry
            lax.fori_loop(0, npt // L, dis_loop, None)
            pltpu.sync_copy(nbuf, deg_sh.at[pl.ds(nb, npt)])
            plsc.subcore_barrier()

            def pass_b(i, carry):
                b = pl.multiple_of(i * EC, EC)
                pltpu.sync_copy(col_h.at[pl.ds(b, EC)], eci)
                pltpu.sync_copy(row_h.at[pl.ds(b, EC)], eri)
                pltpu.sync_copy(w_h.at[pl.ds(b, EC)], ew)
                pltpu.sync_copy(deg_sh.at[eci], eg)

                def mul_loop(j, c2):
                    sl = pl.ds(j * L, L)
                    ew[sl] = ew[sl] * eg[sl]
                    return c2
                lax.fori_loop(0, EC // L, mul_loop, None)
                pltpu.sync_copy(ew, u_sh.at[eri], add=True)
                return carry
            lax.fori_loop(lo, hi, pass_b, None)
            plsc.subcore_barrier()

            pltpu.sync_copy(deg_sh.at[pl.ds(nb, npt)], nbuf)
            pltpu.sync_copy(u_sh.at[pl.ds(nb, npt)], wb.at[pl.ds(0, npt)])

            def c_loop(j, carry):
                sl = pl.ds(j * L, L)
                dv = nbuf[sl]
                nbuf[sl] = dv * (wb[sl] + dv)
                return carry
            lax.fori_loop(0, npt // L, c_loop, None)
            pltpu.sync_copy(nbuf, u_sh.at[pl.ds(nb, npt)])
            plsc.subcore_barrier()

            clo = t * cch_pt
            chi = jnp.minimum(clo + cch_pt, n_cch)

            def pass_c(i, carry):
                b = pl.multiple_of(i * CC, 16)
                pltpu.sync_copy(x_h.at[pl.ds(b, CC)], cxi)
                pltpu.sync_copy(u_sh.at[pl.ds(b, CC)], ccv)
                pltpu.sync_copy(ccv, cc_sh.at[cxi], add=True)
                return carry
            lax.fori_loop(clo, chi, pass_c, None)
            plsc.subcore_barrier()

            pltpu.sync_copy(cc_sh.at[pl.ds(t * spt, spt)], wb)
            pltpu.sync_copy(wb, cc_h.at[pl.ds(t * spt, spt)])

        @pl.when(cid == 1)
        def _core1():
            b = pl.multiple_of(lax.axis_index("s") * bpt, 16)
            pltpu.sync_copy(st_h.at[pl.ds(b, bpt)], sti)
            pltpu.sync_copy(emb_h.at[sti], srows)
            pltpu.sync_copy(srows, x1_h.at[pl.ds(b, bpt)])

    return sc_kernel


def _tc_head(S, B, EMB, HID, A, n_nodes, n_k):
    assert S % n_k == 0
    kblk = S // n_k
    assert kblk % 8 == 0
    inv_n = 1.0 / float(n_nodes)

    def body(cc_ref, emb_ref, x1_ref, gw_ref, gb_ref, w1_ref, b1_ref,
             w2_ref, b2_ref, w3_ref, b3_ref, vw_ref, vb_ref, aw_ref, ab_ref,
             out_ref, acc_ref):
        i = pl.program_id(0)

        @pl.when(i == 0)
        def _():
            acc_ref[...] = jnp.zeros_like(acc_ref)

        acc_ref[...] += jnp.sum(cc_ref[...] * emb_ref[...], axis=0,
                                keepdims=True)

        @pl.when(i == n_k - 1)
        def _():
            s = acc_ref[...] * inv_n
            x2 = jnp.dot(s, gw_ref[...], preferred_element_type=jnp.float32) \
                + gb_ref[...]
            h = (jnp.dot(x1_ref[...], w1_ref[0:EMB, :],
                         preferred_element_type=jnp.float32)
                 + jnp.dot(x2, w1_ref[EMB:, :],
                           preferred_element_type=jnp.float32)
                 + b1_ref[...])
            h = jnp.maximum(h, 0.0)
            h = jnp.maximum(jnp.dot(h, w2_ref[...],
                                    preferred_element_type=jnp.float32)
                            + b2_ref[...], 0.0)
            h = jnp.maximum(jnp.dot(h, w3_ref[...],
                                    preferred_element_type=jnp.float32)
                            + b3_ref[...], 0.0)
            v = jnp.dot(h, vw_ref[...], preferred_element_type=jnp.float32) \
                + vb_ref[...]
            adv = jnp.dot(h, aw_ref[...], preferred_element_type=jnp.float32) \
                + ab_ref[...]
            out_ref[...] = v + adv - jnp.mean(adv, axis=1, keepdims=True)

    const = lambda bs: pl.BlockSpec(bs, lambda i: (0, 0))
    return pl.pallas_call(
        body,
        grid=(n_k,),
        in_specs=[
            pl.BlockSpec((kblk, 1), lambda i: (i, 0)),
            pl.BlockSpec((kblk, EMB), lambda i: (i, 0)),
            const((B, EMB)),
            const((EMB, HID)),
            const((1, HID)),
            const((EMB + HID, HID)),
            const((1, HID)),
            const((HID, HID)),
            const((1, HID)),
            const((HID, HID)),
            const((1, HID)),
            const((HID, 1)),
            const((1, 1)),
            const((HID, A)),
            const((1, A)),
        ],
        out_specs=pl.BlockSpec((B, A), lambda i: (0, 0)),
        out_shape=jax.ShapeDtypeStruct((B, A), jnp.float32),
        scratch_shapes=[pltpu.VMEM((1, EMB), jnp.float32)],
    )


def kernel(state, x, edge_index, edge_weight, emb_table, gcn_W, gcn_b,
           fc1_W, fc1_b, fc2_W, fc2_b, fc3_W, fc3_b, val_W, val_b,
           adv_W, adv_b):
    N = x.shape[0]
    E = edge_weight.shape[0]
    S, EMB = emb_table.shape
    HID = fc2_W.shape[0]
    B = state.shape[0]
    A = adv_W.shape[1]

    row = edge_index[0].astype(jnp.int32)
    col = edge_index[1].astype(jnp.int32)
    xi = x.astype(jnp.int32)
    sti = state[:, 0].astype(jnp.int32)
    w32 = edge_weight.astype(jnp.float32)

    cc_pad, x1 = _sc_coeffs(N, E, S, B, EMB)(
        row, col, w32, xi, sti, emb_table)
    cc2 = cc_pad[:S].reshape(S, 1)

    return _tc_head(S, B, EMB, HID, A, N, 5)(
        cc2, emb_table, x1,
        gcn_W, gcn_b.reshape(1, HID),
        fc1_W, fc1_b.reshape(1, HID),
        fc2_W, fc2_b.reshape(1, HID),
        fc3_W, fc3_b.reshape(1, HID),
        val_W, val_b.reshape(1, 1),
        adv_W, adv_b.reshape(1, A))


# speedup vs baseline: 58.1806x; 3.7918x over previous
import functools

import jax
import jax.numpy as jnp
from jax import lax
from jax.experimental import pallas as pl
from jax.experimental.pallas import tpu as pltpu
from jax.experimental.pallas import tpu_sc as plsc


def _rsqrt_newton(d):
    i = plsc.bitcast(d, jnp.int32)
    i = jnp.int32(0x5F3759DF) - lax.shift_right_logical(i, 1)
    y = plsc.bitcast(i, jnp.float32)
    for _ in range(3):
        y = y * (1.5 - 0.5 * d * y * y)
    return y


def _sc_coeffs(N, E, S, B, EMB):
    L = 16
    NT = 16
    NL = NT * L
    N2 = ((N + NL - 1) // NL) * NL
    S2 = ((S + NL - 1) // NL) * NL
    npt = N2 // NT
    spt = S2 // NT
    EC = 128
    assert E % EC == 0
    n_ech = E // EC
    ech_pt = -(-n_ech // NT)
    G = 16
    CC = 80
    assert N % CC == 0
    n_cch = N // CC
    cch_pt = -(-n_cch // NT)
    bpt = B // NT
    assert B % NT == 0

    mesh = plsc.VectorSubcoreMesh(core_axis_name="c", subcore_axis_name="s")

    @functools.partial(
        pl.kernel,
        out_type=[
            jax.ShapeDtypeStruct((S2,), jnp.float32),
            jax.ShapeDtypeStruct((B, EMB), jnp.float32),
        ],
        mesh=mesh,
        compiler_params=pltpu.CompilerParams(needs_layout_passes=False,
                                             use_tc_tiling_on_sc=False),
        scratch_types=[
            pltpu.VMEM_SHARED((N2,), jnp.float32),
            pltpu.VMEM_SHARED((N2,), jnp.float32),
            pltpu.VMEM_SHARED((S2,), jnp.float32),
            pltpu.VMEM((spt,), jnp.float32),
            pltpu.VMEM((npt,), jnp.float32),
            pltpu.VMEM((G, EC), jnp.int32),
            pltpu.VMEM((G, EC), jnp.int32),
            pltpu.VMEM((G, EC), jnp.int32),
            pltpu.VMEM((G, EC), jnp.int32),
            pltpu.VMEM((G, EC), jnp.float32),
            pltpu.VMEM((G, EC), jnp.float32),
            pltpu.VMEM((G, EC), jnp.float32),
            pltpu.VMEM((1, EC), jnp.int32),
            pltpu.VMEM((1, EC), jnp.int32),
            pltpu.VMEM((1, EC), jnp.float32),
            pltpu.VMEM((1, EC), jnp.float32),
            pltpu.VMEM((CC,), jnp.int32),
            pltpu.VMEM((CC,), jnp.float32),
            pltpu.VMEM((bpt,), jnp.int32),
            pltpu.VMEM((bpt, EMB), jnp.float32),
            pltpu.SemaphoreType.DMA,
            pltpu.SemaphoreType.DMA,
        ],
    )
    def sc_kernel(col2_h, row2_h, w2_h, x_h, st_h, emb_h, cc_h, x1_h,
                  deg_sh, u_sh, cc_sh, wb, nbuf,
                  cb0, cb1, rb0, rb1, wb0, wb1, gb,
                  tci, tri, twv, tgv, cxi, ccv, sti, srows, lsem, ssem):
        cid = lax.axis_index("c")
        sid = lax.axis_index("s")
        cbs, rbs, wbs = (cb0, cb1), (rb0, rb1), (wb0, wb1)

        @pl.when(cid == 0)
        def _core0():
            t = sid
            nb = t * npt

            def zero_loop(j, carry):
                wb[pl.ds(j * L, L)] = jnp.zeros((L,), jnp.float32)
                return carry
            lax.fori_loop(0, spt // L, zero_loop, None)
            pltpu.sync_copy(wb.at[pl.ds(0, npt)], deg_sh.at[pl.ds(nb, npt)])
            pltpu.sync_copy(wb.at[pl.ds(0, npt)], u_sh.at[pl.ds(nb, npt)])
            pltpu.sync_copy(wb, cc_sh.at[pl.ds(t * spt, spt)])
            plsc.subcore_barrier()

            lo = t * ech_pt
            hi = jnp.minimum(lo + ech_pt, n_ech)
            nfull = (hi - lo) // G
            tail_lo = lo + nfull * G

            def fire_a(b, par):
                base = lo + b * G
                pltpu.async_copy(col2_h.at[pl.ds(base, G)], cbs[par], lsem)
                pltpu.async_copy(w2_h.at[pl.ds(base, G)], wbs[par], lsem)

            def wait_a(b, par):
                base = lo + b * G
                pltpu.make_async_copy(
                    col2_h.at[pl.ds(base, G)], cbs[par], lsem).wait()
                pltpu.make_async_copy(
                    w2_h.at[pl.ds(base, G)], wbs[par], lsem).wait()

            @pl.when(nfull > 0)
            def _():
                fire_a(0, 0)

            def body_a(s, carry):
                for par in (0, 1):
                    b = 2 * s + par

                    @pl.when(b < nfull)
                    def _():
                        wait_a(b, par)

                        @pl.when(b + 1 < nfull)
                        def _():
                            fire_a(b + 1, 1 - par)
                        for j in range(G):
                            pltpu.async_copy(wbs[par].at[j],
                                             deg_sh.at[cbs[par].at[j]],
                                             ssem, add=True)
                        for j in range(G):
                            pltpu.make_async_copy(
                                wbs[par].at[j], deg_sh.at[cbs[par].at[j]],
                                ssem).wait()
                return carry
            lax.fori_loop(0, (nfull + 1) // 2, body_a, None)

            def tail_a(i, carry):
                pltpu.sync_copy(col2_h.at[pl.ds(i, 1)], tci)
                pltpu.sync_copy(w2_h.at[pl.ds(i, 1)], twv)
                pltpu.sync_copy(twv.at[0], deg_sh.at[tci.at[0]], add=True)
                return carry
            lax.fori_loop(tail_lo, hi, tail_a, None)
            plsc.subcore_barrier()

            pltpu.sync_copy(deg_sh.at[pl.ds(nb, npt)], nbuf)

            def dis_loop(j, carry):
                sl = pl.ds(j * L, L)
                nbuf[sl] = _rsqrt_newton(nbuf[sl] + 1.0)
                return carry
            lax.fori_loop(0, npt // L, dis_loop, None)
            pltpu.sync_copy(nbuf, deg_sh.at[pl.ds(nb, npt)])
            plsc.subcore_barrier()

            def fire_b(b, par):
                base = lo + b * G
                pltpu.async_copy(col2_h.at[pl.ds(base, G)], cbs[par], lsem)
                pltpu.async_copy(row2_h.at[pl.ds(base, G)], rbs[par], lsem)
                pltpu.async_copy(w2_h.at[pl.ds(base, G)], wbs[par], lsem)

            def wait_b(b, par):
                base = lo + b * G
                pltpu.make_async_copy(
                    col2_h.at[pl.ds(base, G)], cbs[par], lsem).wait()
                pltpu.make_async_copy(
                    row2_h.at[pl.ds(base, G)], rbs[par], lsem).wait()
                pltpu.make_async_copy(
                    w2_h.at[pl.ds(base, G)], wbs[par], lsem).wait()

            @pl.when(nfull > 0)
            def _():
                fire_b(0, 0)

            def body_b(s, carry):
                for par in (0, 1):
                    b = 2 * s + par

                    @pl.when(b < nfull)
                    def _():
                        wait_b(b, par)

                        @pl.when(b + 1 < nfull)
                        def _():
                            fire_b(b + 1, 1 - par)
                        for j in range(G):
                            pltpu.async_copy(deg_sh.at[cbs[par].at[j]],
                                             gb.at[j], ssem)
                        for j in range(G):
                            pltpu.make_async_copy(
                                deg_sh.at[cbs[par].at[j]], gb.at[j],
                                ssem).wait()
                        for j in range(G):
                            for q in range(EC // L):
                                sl = pl.ds(q * L, L)
                                wbs[par][j, sl] = wbs[par][j, sl] * gb[j, sl]
                        for j in range(G):
                            pltpu.async_copy(wbs[par].at[j],
                                             u_sh.at[rbs[par].at[j]],
                                             ssem, add=True)
                        for j in range(G):
                            pltpu.make_async_copy(
                                wbs[par].at[j], u_sh.at[rbs[par].at[j]],
                                ssem).wait()
                return carry
            lax.fori_loop(0, (nfull + 1) // 2, body_b, None)

            def tail_b(i, carry):
                pltpu.sync_copy(col2_h.at[pl.ds(i, 1)], tci)
                pltpu.sync_copy(row2_h.at[pl.ds(i, 1)], tri)
                pltpu.sync_copy(w2_h.at[pl.ds(i, 1)], twv)
                pltpu.sync_copy(deg_sh.at[tci.at[0]], tgv.at[0])
                for q in range(EC // L):
                    sl = pl.ds(q * L, L)
                    twv[0, sl] = twv[0, sl] * tgv[0, sl]
                pltpu.sync_copy(twv.at[0], u_sh.at[tri.at[0]], add=True)
                return carry
            lax.fori_loop(tail_lo, hi, tail_b, None)
            plsc.subcore_barrier()

            pltpu.sync_copy(deg_sh.at[pl.ds(nb, npt)], nbuf)
            pltpu.sync_copy(u_sh.at[pl.ds(nb, npt)], wb.at[pl.ds(0, npt)])

            def c_loop(j, carry):
                sl = pl.ds(j * L, L)
                dv = nbuf[sl]
                nbuf[sl] = dv * (wb[sl] + dv)
                return carry
            lax.fori_loop(0, npt // L, c_loop, None)
            pltpu.sync_copy(nbuf, u_sh.at[pl.ds(nb, npt)])
            plsc.subcore_barrier()

            clo = t * cch_pt
            chi = jnp.minimum(clo + cch_pt, n_cch)

            def pass_c(i, carry):
                b = pl.multiple_of(i * CC, 16)
                pltpu.sync_copy(x_h.at[pl.ds(b, CC)], cxi)
                pltpu.sync_copy(u_sh.at[pl.ds(b, CC)], ccv)
                pltpu.sync_copy(ccv, cc_sh.at[cxi], add=True)
                return carry
            lax.fori_loop(clo, chi, pass_c, None)
            plsc.subcore_barrier()

            pltpu.sync_copy(cc_sh.at[pl.ds(t * spt, spt)], wb)
            pltpu.sync_copy(wb, cc_h.at[pl.ds(t * spt, spt)])

        @pl.when(cid == 1)
        def _core1():
            b = pl.multiple_of(lax.axis_index("s") * bpt, 16)
            pltpu.sync_copy(st_h.at[pl.ds(b, bpt)], sti)
            pltpu.sync_copy(emb_h.at[sti], srows)
            pltpu.sync_copy(srows, x1_h.at[pl.ds(b, bpt)])

    return sc_kernel


def _tc_head(S, B, EMB, HID, A, n_nodes, n_k):
    assert S % n_k == 0
    kblk = S // n_k
    assert kblk % 8 == 0
    inv_n = 1.0 / float(n_nodes)

    def body(cc_ref, emb_ref, x1_ref, gw_ref, gb_ref, w1_ref, b1_ref,
             w2_ref, b2_ref, w3_ref, b3_ref, vw_ref, vb_ref, aw_ref, ab_ref,
             out_ref, acc_ref):
        i = pl.program_id(0)

        @pl.when(i == 0)
        def _():
            acc_ref[...] = jnp.zeros_like(acc_ref)

        acc_ref[...] += jnp.sum(cc_ref[...] * emb_ref[...], axis=0,
                                keepdims=True)

        @pl.when(i == n_k - 1)
        def _():
            s = acc_ref[...] * inv_n
            x2 = jnp.dot(s, gw_ref[...], preferred_element_type=jnp.float32) \
                + gb_ref[...]
            h = (jnp.dot(x1_ref[...], w1_ref[0:EMB, :],
                         preferred_element_type=jnp.float32)
                 + jnp.dot(x2, w1_ref[EMB:, :],
                           preferred_element_type=jnp.float32)
                 + b1_ref[...])
            h = jnp.maximum(h, 0.0)
            h = jnp.maximum(jnp.dot(h, w2_ref[...],
                                    preferred_element_type=jnp.float32)
                            + b2_ref[...], 0.0)
            h = jnp.maximum(jnp.dot(h, w3_ref[...],
                                    preferred_element_type=jnp.float32)
                            + b3_ref[...], 0.0)
            v = jnp.dot(h, vw_ref[...], preferred_element_type=jnp.float32) \
                + vb_ref[...]
            adv = jnp.dot(h, aw_ref[...], preferred_element_type=jnp.float32) \
                + ab_ref[...]
            out_ref[...] = v + adv - jnp.mean(adv, axis=1, keepdims=True)

    const = lambda bs: pl.BlockSpec(bs, lambda i: (0, 0))
    return pl.pallas_call(
        body,
        grid=(n_k,),
        in_specs=[
            pl.BlockSpec((kblk, 1), lambda i: (i, 0)),
            pl.BlockSpec((kblk, EMB), lambda i: (i, 0)),
            const((B, EMB)),
            const((EMB, HID)),
            const((1, HID)),
            const((EMB + HID, HID)),
            const((1, HID)),
            const((HID, HID)),
            const((1, HID)),
            const((HID, HID)),
            const((1, HID)),
            const((HID, 1)),
            const((1, 1)),
            const((HID, A)),
            const((1, A)),
        ],
        out_specs=pl.BlockSpec((B, A), lambda i: (0, 0)),
        out_shape=jax.ShapeDtypeStruct((B, A), jnp.float32),
        scratch_shapes=[pltpu.VMEM((1, EMB), jnp.float32)],
    )


def kernel(state, x, edge_index, edge_weight, emb_table, gcn_W, gcn_b,
           fc1_W, fc1_b, fc2_W, fc2_b, fc3_W, fc3_b, val_W, val_b,
           adv_W, adv_b):
    N = x.shape[0]
    E = edge_weight.shape[0]
    S, EMB = emb_table.shape
    HID = fc2_W.shape[0]
    B = state.shape[0]
    A = adv_W.shape[1]

    EC = 128
    row2 = edge_index[0].astype(jnp.int32).reshape(E // EC, EC)
    col2 = edge_index[1].astype(jnp.int32).reshape(E // EC, EC)
    w2 = edge_weight.astype(jnp.float32).reshape(E // EC, EC)
    xi = x.astype(jnp.int32)
    sti = state[:, 0].astype(jnp.int32)

    cc_pad, x1 = _sc_coeffs(N, E, S, B, EMB)(
        col2, row2, w2, xi, sti, emb_table)
    cc2 = cc_pad[:S].reshape(S, 1)

    return _tc_head(S, B, EMB, HID, A, N, 5)(
        cc2, emb_table, x1,
        gcn_W, gcn_b.reshape(1, HID),
        fc1_W, fc1_b.reshape(1, HID),
        fc2_W, fc2_b.reshape(1, HID),
        fc3_W, fc3_b.reshape(1, HID),
        val_W, val_b.reshape(1, 1),
        adv_W, adv_b.reshape(1, A))


# speedup vs baseline: 63.3775x; 1.0893x over previous
import functools

import jax
import jax.numpy as jnp
from jax import lax
from jax.experimental import pallas as pl
from jax.experimental.pallas import tpu as pltpu
from jax.experimental.pallas import tpu_sc as plsc


def _rsqrt_newton(d):
    i = plsc.bitcast(d, jnp.int32)
    i = jnp.int32(0x5F3759DF) - lax.shift_right_logical(i, 1)
    y = plsc.bitcast(i, jnp.float32)
    for _ in range(3):
        y = y * (1.5 - 0.5 * d * y * y)
    return y


def _sc_coeffs(N, E, S, B, EMB):
    L = 16
    NT = 16
    NL = NT * L
    N2 = ((N + NL - 1) // NL) * NL
    S2 = ((S + NL - 1) // NL) * NL
    npt = N2 // NT
    spt = S2 // NT
    EC = 128
    assert E % EC == 0
    n_ech = E // EC
    ech_pt = -(-n_ech // NT)
    EPT = ech_pt * EC
    G = 16
    GE = G * EC
    CC = 80
    assert N % CC == 0
    n_cch = N // CC
    cch_pt = -(-n_cch // NT)
    bpt = B // NT
    assert B % NT == 0

    mesh = plsc.VectorSubcoreMesh(core_axis_name="c", subcore_axis_name="s")

    @functools.partial(
        pl.kernel,
        out_type=[
            jax.ShapeDtypeStruct((S2,), jnp.float32),
            jax.ShapeDtypeStruct((B, EMB), jnp.float32),
        ],
        mesh=mesh,
        compiler_params=pltpu.CompilerParams(needs_layout_passes=False,
                                             use_tc_tiling_on_sc=False),
        scratch_types=[
            pltpu.VMEM_SHARED((N2,), jnp.float32),
            pltpu.VMEM_SHARED((N2,), jnp.float32),
            pltpu.VMEM_SHARED((S2,), jnp.float32),
            pltpu.VMEM((spt,), jnp.float32),
            pltpu.VMEM((npt,), jnp.float32),
            pltpu.VMEM((EPT,), jnp.int32),
            pltpu.VMEM((GE,), jnp.float32),
            pltpu.VMEM((GE,), jnp.float32),
            pltpu.VMEM((GE,), jnp.int32),
            pltpu.VMEM((GE,), jnp.int32),
            pltpu.VMEM((GE,), jnp.float32),
            pltpu.VMEM((CC,), jnp.int32),
            pltpu.VMEM((CC,), jnp.float32),
            pltpu.VMEM((bpt,), jnp.int32),
            pltpu.VMEM((bpt, EMB), jnp.float32),
            pltpu.SemaphoreType.DMA,
            pltpu.SemaphoreType.DMA,
        ],
    )
    def sc_kernel(ei_h, w_h, x_h, st_h, emb_h, cc_h, x1_h,
                  deg_sh, u_sh, cc_sh, wb, nbuf, colv, wv0, wv1,
                  rb0, rb1, gb, cxi, ccv, sti, srows, lsem, ssem):
        cid = lax.axis_index("c")
        sid = lax.axis_index("s")
        rbs = (rb0, rb1)
        wvs = (wv0, wv1)

        @pl.when(cid == 0)
        def _core0():
            t = sid
            nb = t * npt

            def zero_loop(j, carry):
                wb[pl.ds(j * L, L)] = jnp.zeros((L,), jnp.float32)
                return carry
            lax.fori_loop(0, spt // L, zero_loop, None)
            pltpu.sync_copy(wb.at[pl.ds(0, npt)], deg_sh.at[pl.ds(nb, npt)])
            pltpu.sync_copy(wb.at[pl.ds(0, npt)], u_sh.at[pl.ds(nb, npt)])
            pltpu.sync_copy(wb, cc_sh.at[pl.ds(t * spt, spt)])
            plsc.subcore_barrier()

            lo = t * ech_pt
            hi = jnp.minimum(lo + ech_pt, n_ech)
            nch = hi - lo
            nfull = nch // G
            tail_lo = nfull * G

            def fire_a(b, par):
                src = pl.multiple_of((lo + b * G) * EC, EC)
                dst = pl.multiple_of(b * GE, EC)
                pltpu.async_copy(ei_h.at[1, pl.ds(src, GE)],
                                 colv.at[pl.ds(dst, GE)], lsem)
                pltpu.async_copy(w_h.at[pl.ds(src, GE)], wvs[par], lsem)

            def wait_a(b, par):
                src = pl.multiple_of((lo + b * G) * EC, EC)
                dst = pl.multiple_of(b * GE, EC)
                pltpu.make_async_copy(ei_h.at[1, pl.ds(src, GE)],
                                      colv.at[pl.ds(dst, GE)], lsem).wait()
                pltpu.make_async_copy(w_h.at[pl.ds(src, GE)],
                                      wvs[par], lsem).wait()

            @pl.when(nfull > 0)
            def _():
                fire_a(0, 0)

            def body_a(s, carry):
                for par in (0, 1):
                    b = 2 * s + par

                    @pl.when(b < nfull)
                    def _():
                        wait_a(b, par)

                        @pl.when(b + 1 < nfull)
                        def _():
                            fire_a(b + 1, 1 - par)
                        base = pl.multiple_of(b * GE, EC)
                        for j in range(G):
                            sl = pl.ds(base + j * EC, EC)
                            vs = pl.ds(j * EC, EC)
                            pltpu.async_copy(wvs[par].at[vs],
                                             deg_sh.at[colv.at[sl]],
                                             ssem, add=True)
                        for j in range(G):
                            sl = pl.ds(base + j * EC, EC)
                            vs = pl.ds(j * EC, EC)
                            pltpu.make_async_copy(wvs[par].at[vs],
                                                  deg_sh.at[colv.at[sl]],
                                                  ssem).wait()
                return carry
            lax.fori_loop(0, (nfull + 1) // 2, body_a, None)

            def tail_a(c, carry):
                src = pl.multiple_of((lo + c) * EC, EC)
                dst = pl.multiple_of(c * EC, EC)
                sl = pl.ds(dst, EC)
                vs = pl.ds(0, EC)
                pltpu.sync_copy(ei_h.at[1, pl.ds(src, EC)], colv.at[sl])
                pltpu.sync_copy(w_h.at[pl.ds(src, EC)], wv0.at[vs])
                pltpu.sync_copy(wv0.at[vs], deg_sh.at[colv.at[sl]], add=True)
                return carry
            lax.fori_loop(tail_lo, nch, tail_a, None)
            plsc.subcore_barrier()

            pltpu.sync_copy(deg_sh.at[pl.ds(nb, npt)], nbuf)

            def dis_loop(j, carry):
                sl = pl.ds(j * L, L)
                nbuf[sl] = _rsqrt_newton(nbuf[sl] + 1.0)
                return carry
            lax.fori_loop(0, npt // L, dis_loop, None)
            pltpu.sync_copy(nbuf, deg_sh.at[pl.ds(nb, npt)])
            plsc.subcore_barrier()

            def fire_b(b, par):
                src = pl.multiple_of((lo + b * G) * EC, EC)
                pltpu.async_copy(ei_h.at[0, pl.ds(src, GE)], rbs[par], lsem)
                pltpu.async_copy(w_h.at[pl.ds(src, GE)], wvs[par], lsem)

            def wait_b(b, par):
                src = pl.multiple_of((lo + b * G) * EC, EC)
                pltpu.make_async_copy(ei_h.at[0, pl.ds(src, GE)],
                                      rbs[par], lsem).wait()
                pltpu.make_async_copy(w_h.at[pl.ds(src, GE)],
                                      wvs[par], lsem).wait()

            @pl.when(nfull > 0)
            def _():
                fire_b(0, 0)

            def body_b(s, carry):
                for par in (0, 1):
                    b = 2 * s + par

                    @pl.when(b < nfull)
                    def _():
                        wait_b(b, par)

                        @pl.when(b + 1 < nfull)
                        def _():
                            fire_b(b + 1, 1 - par)
                        base = pl.multiple_of(b * GE, EC)
                        for j in range(G):
                            sl = pl.ds(base + j * EC, EC)
                            gsl = pl.ds(j * EC, EC)
                            pltpu.async_copy(deg_sh.at[colv.at[sl]],
                                             gb.at[gsl], ssem)
                        for j in range(G):
                            sl = pl.ds(base + j * EC, EC)
                            gsl = pl.ds(j * EC, EC)
                            pltpu.make_async_copy(deg_sh.at[colv.at[sl]],
                                                  gb.at[gsl], ssem).wait()
                        for q in range(GE // L):
                            vq = pl.ds(q * L, L)
                            wvs[par][vq] = wvs[par][vq] * gb[vq]
                        for j in range(G):
                            vs = pl.ds(j * EC, EC)
                            pltpu.async_copy(wvs[par].at[vs],
                                             u_sh.at[rbs[par].at[vs]],
                                             ssem, add=True)
                        for j in range(G):
                            vs = pl.ds(j * EC, EC)
                            pltpu.make_async_copy(wvs[par].at[vs],
                                                  u_sh.at[rbs[par].at[vs]],
                                                  ssem).wait()
                return carry
            lax.fori_loop(0, (nfull + 1) // 2, body_b, None)

            def tail_b(c, carry):
                src = pl.multiple_of((lo + c) * EC, EC)
                dst = pl.multiple_of(c * EC, EC)
                sl = pl.ds(dst, EC)
                vs = pl.ds(0, EC)
                pltpu.sync_copy(ei_h.at[0, pl.ds(src, EC)], rb0.at[vs])
                pltpu.sync_copy(w_h.at[pl.ds(src, EC)], wv0.at[vs])
                pltpu.sync_copy(deg_sh.at[colv.at[sl]], gb.at[vs])
                for q in range(EC // L):
                    vq = pl.ds(q * L, L)
                    wv0[vq] = wv0[vq] * gb[vq]
                pltpu.sync_copy(wv0.at[vs], u_sh.at[rb0.at[vs]], add=True)
                return carry
            lax.fori_loop(tail_lo, nch, tail_b, None)
            plsc.subcore_barrier()

            pltpu.sync_copy(deg_sh.at[pl.ds(nb, npt)], nbuf)
            pltpu.sync_copy(u_sh.at[pl.ds(nb, npt)], wb.at[pl.ds(0, npt)])

            def c_loop(j, carry):
                sl = pl.ds(j * L, L)
                dv = nbuf[sl]
                nbuf[sl] = dv * (wb[sl] + dv)
                return carry
            lax.fori_loop(0, npt // L, c_loop, None)
            pltpu.sync_copy(nbuf, u_sh.at[pl.ds(nb, npt)])
            plsc.subcore_barrier()

            clo = t * cch_pt
            chi = jnp.minimum(clo + cch_pt, n_cch)

            def pass_c(i, carry):
                b = pl.multiple_of(i * CC, 16)
                pltpu.sync_copy(x_h.at[pl.ds(b, CC)], cxi)
                pltpu.sync_copy(u_sh.at[pl.ds(b, CC)], ccv)
                pltpu.sync_copy(ccv, cc_sh.at[cxi], add=True)
                return carry
            lax.fori_loop(clo, chi, pass_c, None)
            plsc.subcore_barrier()

            pltpu.sync_copy(cc_sh.at[pl.ds(t * spt, spt)], wb)
            pltpu.sync_copy(wb, cc_h.at[pl.ds(t * spt, spt)])

        @pl.when(cid == 1)
        def _core1():
            b = pl.multiple_of(lax.axis_index("s") * bpt, 16)
            pltpu.sync_copy(st_h.at[pl.ds(b, bpt)], sti)
            pltpu.sync_copy(emb_h.at[sti], srows)
            pltpu.sync_copy(srows, x1_h.at[pl.ds(b, bpt)])

    return sc_kernel


def _tc_head(S, B, EMB, HID, A, n_nodes, n_k):
    assert S % n_k == 0
    kblk = S // n_k
    assert kblk % 8 == 0
    inv_n = 1.0 / float(n_nodes)

    def body(cc_ref, emb_ref, x1_ref, gw_ref, gb_ref, w1_ref, b1_ref,
             w2_ref, b2_ref, w3_ref, b3_ref, vw_ref, vb_ref, aw_ref, ab_ref,
             out_ref, acc_ref):
        i = pl.program_id(0)

        @pl.when(i == 0)
        def _():
            acc_ref[...] = jnp.zeros_like(acc_ref)

        acc_ref[...] += jnp.sum(cc_ref[...] * emb_ref[...], axis=0,
                                keepdims=True)

        @pl.when(i == n_k - 1)
        def _():
            s = acc_ref[...] * inv_n
            x2 = jnp.dot(s, gw_ref[...], preferred_element_type=jnp.float32) \
                + gb_ref[...]
            h = (jnp.dot(x1_ref[...], w1_ref[0:EMB, :],
                         preferred_element_type=jnp.float32)
                 + jnp.dot(x2, w1_ref[EMB:, :],
                           preferred_element_type=jnp.float32)
                 + b1_ref[...])
            h = jnp.maximum(h, 0.0)
            h = jnp.maximum(jnp.dot(h, w2_ref[...],
                                    preferred_element_type=jnp.float32)
                            + b2_ref[...], 0.0)
            h = jnp.maximum(jnp.dot(h, w3_ref[...],
                                    preferred_element_type=jnp.float32)
                            + b3_ref[...], 0.0)
            v = jnp.dot(h, vw_ref[...], preferred_element_type=jnp.float32) \
                + vb_ref[...]
            adv = jnp.dot(h, aw_ref[...], preferred_element_type=jnp.float32) \
                + ab_ref[...]
            out_ref[...] = v + adv - jnp.mean(adv, axis=1, keepdims=True)

    const = lambda bs: pl.BlockSpec(bs, lambda i: (0, 0))
    return pl.pallas_call(
        body,
        grid=(n_k,),
        in_specs=[
            pl.BlockSpec((kblk, 1), lambda i: (i, 0)),
            pl.BlockSpec((kblk, EMB), lambda i: (i, 0)),
            const((B, EMB)),
            const((EMB, HID)),
            const((1, HID)),
            const((EMB + HID, HID)),
            const((1, HID)),
            const((HID, HID)),
            const((1, HID)),
            const((HID, HID)),
            const((1, HID)),
            const((HID, 1)),
            const((1, 1)),
            const((HID, A)),
            const((1, A)),
        ],
        out_specs=pl.BlockSpec((B, A), lambda i: (0, 0)),
        out_shape=jax.ShapeDtypeStruct((B, A), jnp.float32),
        scratch_shapes=[pltpu.VMEM((1, EMB), jnp.float32)],
    )


def kernel(state, x, edge_index, edge_weight, emb_table, gcn_W, gcn_b,
           fc1_W, fc1_b, fc2_W, fc2_b, fc3_W, fc3_b, val_W, val_b,
           adv_W, adv_b):
    N = x.shape[0]
    E = edge_weight.shape[0]
    S, EMB = emb_table.shape
    HID = fc2_W.shape[0]
    B = state.shape[0]
    A = adv_W.shape[1]

    ei = edge_index.astype(jnp.int32)
    w32 = edge_weight.astype(jnp.float32)
    xi = x.astype(jnp.int32)
    sti = state[:, 0].astype(jnp.int32)

    cc_pad, x1 = _sc_coeffs(N, E, S, B, EMB)(
        ei, w32, xi, sti, emb_table)
    cc2 = cc_pad[:S].reshape(S, 1)

    return _tc_head(S, B, EMB, HID, A, N, 5)(
        cc2, emb_table, x1,
        gcn_W, gcn_b.reshape(1, HID),
        fc1_W, fc1_b.reshape(1, HID),
        fc2_W, fc2_b.reshape(1, HID),
        fc3_W, fc3_b.reshape(1, HID),
        val_W, val_b.reshape(1, 1),
        adv_W, adv_b.reshape(1, A))


# speedup vs baseline: 76.3231x; 1.2043x over previous
import functools

import jax
import jax.numpy as jnp
from jax import lax
from jax.experimental import pallas as pl
from jax.experimental.pallas import tpu as pltpu
from jax.experimental.pallas import tpu_sc as plsc


def _rsqrt_newton(d):
    i = plsc.bitcast(d, jnp.int32)
    i = jnp.int32(0x5F3759DF) - lax.shift_right_logical(i, 1)
    y = plsc.bitcast(i, jnp.float32)
    for _ in range(3):
        y = y * (1.5 - 0.5 * d * y * y)
    return y


def _sc_coeffs(N, E, S, B, EMB):
    L = 16
    NT = 16
    NL = NT * L
    N2 = ((N + NL - 1) // NL) * NL
    S2 = ((S + NL - 1) // NL) * NL
    npt = N2 // NT
    spt = S2 // NT
    EC = 128
    assert E % EC == 0
    n_ech = E // EC
    ech_pt = -(-n_ech // NT)
    EPT = ech_pt * EC
    G = 16
    GE = G * EC
    CC = 80
    assert N % CC == 0
    n_cch = N // CC
    cch_pt = -(-n_cch // NT)
    bpt = B // NT
    assert B % NT == 0

    mesh = plsc.VectorSubcoreMesh(core_axis_name="c", subcore_axis_name="s")

    @functools.partial(
        pl.kernel,
        out_type=[
            jax.ShapeDtypeStruct((S2,), jnp.float32),
            jax.ShapeDtypeStruct((B, EMB), jnp.float32),
        ],
        mesh=mesh,
        compiler_params=pltpu.CompilerParams(needs_layout_passes=False,
                                             use_tc_tiling_on_sc=False),
        scratch_types=[
            pltpu.VMEM_SHARED((N2,), jnp.float32),
            pltpu.VMEM_SHARED((N2,), jnp.float32),
            pltpu.VMEM_SHARED((S2,), jnp.float32),
            pltpu.VMEM((spt,), jnp.float32),
            pltpu.VMEM((npt,), jnp.float32),
            pltpu.VMEM((EPT,), jnp.int32),
            pltpu.VMEM((GE,), jnp.float32),
            pltpu.VMEM((GE,), jnp.float32),
            pltpu.VMEM((GE,), jnp.int32),
            pltpu.VMEM((GE,), jnp.int32),
            pltpu.VMEM((GE,), jnp.float32),
            pltpu.VMEM((CC,), jnp.int32),
            pltpu.VMEM((CC,), jnp.float32),
            pltpu.VMEM((bpt,), jnp.int32),
            pltpu.VMEM((bpt, EMB), jnp.float32),
            pltpu.SemaphoreType.DMA,
            pltpu.SemaphoreType.DMA,
        ],
    )
    def sc_kernel(ei_h, w_h, x_h, st_h, emb_h, cc_h, x1_h,
                  deg_sh, u_sh, cc_sh, wb, nbuf, colv, wv0, wv1,
                  rb0, rb1, gb, cxi, ccv, sti, srows, lsem, ssem):
        cid = lax.axis_index("c")
        sid = lax.axis_index("s")
        rbs = (rb0, rb1)
        wvs = (wv0, wv1)

        @pl.when(cid == 0)
        def _core0():
            t = sid
            nb = t * npt

            def zero_loop(j, carry):
                wb[pl.ds(j * L, L)] = jnp.zeros((L,), jnp.float32)
                return carry
            lax.fori_loop(0, spt // L, zero_loop, None)
            pltpu.sync_copy(wb.at[pl.ds(0, npt)], deg_sh.at[pl.ds(nb, npt)])
            pltpu.sync_copy(wb.at[pl.ds(0, npt)], u_sh.at[pl.ds(nb, npt)])
            pltpu.sync_copy(wb, cc_sh.at[pl.ds(t * spt, spt)])
            plsc.subcore_barrier()

            lo = t * ech_pt
            hi = jnp.minimum(lo + ech_pt, n_ech)
            nch = hi - lo
            nfull = nch // G
            tail_lo = nfull * G

            def fire_a(b, par):
                src = pl.multiple_of((lo + b * G) * EC, EC)
                dst = pl.multiple_of(b * GE, EC)
                pltpu.async_copy(ei_h.at[1, pl.ds(src, GE)],
                                 colv.at[pl.ds(dst, GE)], lsem)
                pltpu.async_copy(w_h.at[pl.ds(src, GE)], wvs[par], lsem)

            def wait_a(b, par):
                src = pl.multiple_of((lo + b * G) * EC, EC)
                dst = pl.multiple_of(b * GE, EC)
                pltpu.make_async_copy(ei_h.at[1, pl.ds(src, GE)],
                                      colv.at[pl.ds(dst, GE)], lsem).wait()
                pltpu.make_async_copy(w_h.at[pl.ds(src, GE)],
                                      wvs[par], lsem).wait()

            @pl.when(nfull > 0)
            def _():
                fire_a(0, 0)

            def body_a(s, carry):
                for par in (0, 1):
                    b = 2 * s + par

                    @pl.when(b < nfull)
                    def _():
                        wait_a(b, par)

                        @pl.when(b + 1 < nfull)
                        def _():
                            fire_a(b + 1, 1 - par)
                        base = pl.multiple_of(b * GE, EC)
                        sl = pl.ds(base, GE)
                        pltpu.sync_copy(wvs[par], deg_sh.at[colv.at[sl]],
                                        add=True)
                return carry
            lax.fori_loop(0, (nfull + 1) // 2, body_a, None)

            def tail_a(c, carry):
                src = pl.multiple_of((lo + c) * EC, EC)
                dst = pl.multiple_of(c * EC, EC)
                sl = pl.ds(dst, EC)
                vs = pl.ds(0, EC)
                pltpu.sync_copy(ei_h.at[1, pl.ds(src, EC)], colv.at[sl])
                pltpu.sync_copy(w_h.at[pl.ds(src, EC)], wv0.at[vs])
                pltpu.sync_copy(wv0.at[vs], deg_sh.at[colv.at[sl]], add=True)
                return carry
            lax.fori_loop(tail_lo, nch, tail_a, None)
            plsc.subcore_barrier()

            pltpu.sync_copy(deg_sh.at[pl.ds(nb, npt)], nbuf)

            def dis_loop(j, carry):
                sl = pl.ds(j * L, L)
                nbuf[sl] = _rsqrt_newton(nbuf[sl] + 1.0)
                return carry
            lax.fori_loop(0, npt // L, dis_loop, None)
            pltpu.sync_copy(nbuf, deg_sh.at[pl.ds(nb, npt)])
            plsc.subcore_barrier()

            def fire_b(b, par):
                src = pl.multiple_of((lo + b * G) * EC, EC)
                pltpu.async_copy(ei_h.at[0, pl.ds(src, GE)], rbs[par], lsem)
                pltpu.async_copy(w_h.at[pl.ds(src, GE)], wvs[par], lsem)

            def wait_b(b, par):
                src = pl.multiple_of((lo + b * G) * EC, EC)
                pltpu.make_async_copy(ei_h.at[0, pl.ds(src, GE)],
                                      rbs[par], lsem).wait()
                pltpu.make_async_copy(w_h.at[pl.ds(src, GE)],
                                      wvs[par], lsem).wait()

            @pl.when(nfull > 0)
            def _():
                fire_b(0, 0)

            def body_b(s, carry):
                for par in (0, 1):
                    b = 2 * s + par

                    @pl.when(b < nfull)
                    def _():
                        wait_b(b, par)

                        @pl.when(b + 1 < nfull)
                        def _():
                            fire_b(b + 1, 1 - par)
                        base = pl.multiple_of(b * GE, EC)
                        sl = pl.ds(base, GE)
                        pltpu.sync_copy(deg_sh.at[colv.at[sl]], gb)
                        for q in range(GE // L):
                            vq = pl.ds(q * L, L)
                            wvs[par][vq] = wvs[par][vq] * gb[vq]
                        pltpu.sync_copy(wvs[par], u_sh.at[rbs[par]],
                                        add=True)
                return carry
            lax.fori_loop(0, (nfull + 1) // 2, body_b, None)

            def tail_b(c, carry):
                src = pl.multiple_of((lo + c) * EC, EC)
                dst = pl.multiple_of(c * EC, EC)
                sl = pl.ds(dst, EC)
                vs = pl.ds(0, EC)
                pltpu.sync_copy(ei_h.at[0, pl.ds(src, EC)], rb0.at[vs])
                pltpu.sync_copy(w_h.at[pl.ds(src, EC)], wv0.at[vs])
                pltpu.sync_copy(deg_sh.at[colv.at[sl]], gb.at[vs])
                for q in range(EC // L):
                    vq = pl.ds(q * L, L)
                    wv0[vq] = wv0[vq] * gb[vq]
                pltpu.sync_copy(wv0.at[vs], u_sh.at[rb0.at[vs]], add=True)
                return carry
            lax.fori_loop(tail_lo, nch, tail_b, None)
            plsc.subcore_barrier()

            pltpu.sync_copy(deg_sh.at[pl.ds(nb, npt)], nbuf)
            pltpu.sync_copy(u_sh.at[pl.ds(nb, npt)], wb.at[pl.ds(0, npt)])

            def c_loop(j, carry):
                sl = pl.ds(j * L, L)
                dv = nbuf[sl]
                nbuf[sl] = dv * (wb[sl] + dv)
                return carry
            lax.fori_loop(0, npt // L, c_loop, None)
            pltpu.sync_copy(nbuf, u_sh.at[pl.ds(nb, npt)])
            plsc.subcore_barrier()

            clo = t * cch_pt
            chi = jnp.minimum(clo + cch_pt, n_cch)

            def pass_c(i, carry):
                b = pl.multiple_of(i * CC, 16)
                pltpu.sync_copy(x_h.at[pl.ds(b, CC)], cxi)
                pltpu.sync_copy(u_sh.at[pl.ds(b, CC)], ccv)
                pltpu.sync_copy(ccv, cc_sh.at[cxi], add=True)
                return carry
            lax.fori_loop(clo, chi, pass_c, None)
            plsc.subcore_barrier()

            pltpu.sync_copy(cc_sh.at[pl.ds(t * spt, spt)], wb)
            pltpu.sync_copy(wb, cc_h.at[pl.ds(t * spt, spt)])

        @pl.when(cid == 1)
        def _core1():
            b = pl.multiple_of(lax.axis_index("s") * bpt, 16)
            pltpu.sync_copy(st_h.at[pl.ds(b, bpt)], sti)
            pltpu.sync_copy(emb_h.at[sti], srows)
            pltpu.sync_copy(srows, x1_h.at[pl.ds(b, bpt)])

    return sc_kernel


def _tc_head(S, S2, B, EMB, HID, A, n_nodes):
    inv_n = 1.0 / float(n_nodes)

    def body(cc_ref, emb_ref, x1_ref, gw_ref, gb_ref, w1_ref, b1_ref,
             w2_ref, b2_ref, w3_ref, b3_ref, vw_ref, vb_ref, aw_ref, ab_ref,
             out_ref):
        cv = cc_ref[pl.ds(0, S)].reshape(1, S)
        acc = jnp.dot(cv, emb_ref[...], preferred_element_type=jnp.float32)
        s = acc * inv_n
        x2 = jnp.dot(s, gw_ref[...], preferred_element_type=jnp.float32) \
            + gb_ref[...]
        h = (jnp.dot(x1_ref[...], w1_ref[0:EMB, :],
                     preferred_element_type=jnp.float32)
             + jnp.dot(x2, w1_ref[EMB:, :],
                       preferred_element_type=jnp.float32)
             + b1_ref[...])
        h = jnp.maximum(h, 0.0)
        h = jnp.maximum(jnp.dot(h, w2_ref[...],
                                preferred_element_type=jnp.float32)
                        + b2_ref[...], 0.0)
        h = jnp.maximum(jnp.dot(h, w3_ref[...],
                                preferred_element_type=jnp.float32)
                        + b3_ref[...], 0.0)
        v = jnp.dot(h, vw_ref[...], preferred_element_type=jnp.float32) \
            + vb_ref[...]
        adv = jnp.dot(h, aw_ref[...], preferred_element_type=jnp.float32) \
            + ab_ref[...]
        out_ref[...] = v + adv - jnp.mean(adv, axis=1, keepdims=True)

    return pl.pallas_call(
        body,
        out_shape=jax.ShapeDtypeStruct((B, A), jnp.float32),
    )


def kernel(state, x, edge_index, edge_weight, emb_table, gcn_W, gcn_b,
           fc1_W, fc1_b, fc2_W, fc2_b, fc3_W, fc3_b, val_W, val_b,
           adv_W, adv_b):
    N = x.shape[0]
    E = edge_weight.shape[0]
    S, EMB = emb_table.shape
    HID = fc2_W.shape[0]
    B = state.shape[0]
    A = adv_W.shape[1]

    ei = edge_index.astype(jnp.int32)
    w32 = edge_weight.astype(jnp.float32)
    xi = x.astype(jnp.int32)
    sti = state[:, 0].astype(jnp.int32)

    cc_pad, x1 = _sc_coeffs(N, E, S, B, EMB)(
        ei, w32, xi, sti, emb_table)
    S2 = cc_pad.shape[0]

    return _tc_head(S, S2, B, EMB, HID, A, N)(
        cc_pad, emb_table, x1,
        gcn_W, gcn_b.reshape(1, HID),
        fc1_W, fc1_b.reshape(1, HID),
        fc2_W, fc2_b.reshape(1, HID),
        fc3_W, fc3_b.reshape(1, HID),
        val_W, val_b.reshape(1, 1),
        adv_W, adv_b.reshape(1, A))


# speedup vs baseline: 78.5265x; 1.0289x over previous
import functools

import jax
import jax.numpy as jnp
from jax import lax
from jax.experimental import pallas as pl
from jax.experimental.pallas import tpu as pltpu
from jax.experimental.pallas import tpu_sc as plsc


def _rsqrt_newton(d):
    i = plsc.bitcast(d, jnp.int32)
    i = jnp.int32(0x5F3759DF) - lax.shift_right_logical(i, 1)
    y = plsc.bitcast(i, jnp.float32)
    for _ in range(3):
        y = y * (1.5 - 0.5 * d * y * y)
    return y


def _sc_coeffs(N, E, S, B, EMB):
    L = 16
    NT = 16
    NL = NT * L
    N2 = ((N + NL - 1) // NL) * NL
    S2 = ((S + NL - 1) // NL) * NL
    npt = N2 // NT
    spt = S2 // NT
    EC = 128
    assert E % EC == 0
    n_ech = E // EC
    ech_pt = -(-n_ech // NT)
    EPT = ech_pt * EC
    G = 16
    GE = G * EC
    CC = 80
    assert N % CC == 0
    n_cch = N // CC
    cch_pt = -(-n_cch // NT)
    bpt = B // NT
    assert B % NT == 0

    mesh = plsc.VectorSubcoreMesh(core_axis_name="c", subcore_axis_name="s")

    @functools.partial(
        pl.kernel,
        out_type=jax.ShapeDtypeStruct((S2,), jnp.float32),
        mesh=mesh,
        compiler_params=pltpu.CompilerParams(needs_layout_passes=False,
                                             use_tc_tiling_on_sc=True),
        scratch_types=[
            pltpu.VMEM_SHARED((N2,), jnp.float32),
            pltpu.VMEM_SHARED((N2,), jnp.float32),
            pltpu.VMEM_SHARED((S2,), jnp.float32),
            pltpu.VMEM((spt,), jnp.float32),
            pltpu.VMEM((npt,), jnp.float32),
            pltpu.VMEM((EPT,), jnp.int32),
            pltpu.VMEM((GE,), jnp.float32),
            pltpu.VMEM((GE,), jnp.float32),
            pltpu.VMEM((GE,), jnp.int32),
            pltpu.VMEM((GE,), jnp.int32),
            pltpu.VMEM((GE,), jnp.float32),
            pltpu.VMEM((CC,), jnp.int32),
            pltpu.VMEM((CC,), jnp.float32),
            pltpu.SemaphoreType.DMA,
            pltpu.SemaphoreType.DMA,
        ],
    )
    def sc_kernel(ei_h, w_h, x_h, cc_h,
                  deg_sh, u_sh, cc_sh, wb, nbuf, colv, wv0, wv1,
                  rb0, rb1, gb, cxi, ccv, lsem, ssem):
        cid = lax.axis_index("c")
        sid = lax.axis_index("s")
        rbs = (rb0, rb1)
        wvs = (wv0, wv1)

        @pl.when(cid == 0)
        def _core0():
            t = sid
            nb = t * npt

            def zero_loop(j, carry):
                wb[pl.ds(j * L, L)] = jnp.zeros((L,), jnp.float32)
                return carry
            lax.fori_loop(0, spt // L, zero_loop, None)
            pltpu.sync_copy(wb.at[pl.ds(0, npt)], deg_sh.at[pl.ds(nb, npt)])
            pltpu.sync_copy(wb.at[pl.ds(0, npt)], u_sh.at[pl.ds(nb, npt)])
            pltpu.sync_copy(wb, cc_sh.at[pl.ds(t * spt, spt)])
            plsc.subcore_barrier()

            lo = t * ech_pt
            hi = jnp.minimum(lo + ech_pt, n_ech)
            nch = hi - lo
            nfull = nch // G
            tail_lo = nfull * G

            def fire_a(b, par):
                src = pl.multiple_of((lo + b * G) * EC, EC)
                dst = pl.multiple_of(b * GE, EC)
                pltpu.async_copy(ei_h.at[1, pl.ds(src, GE)],
                                 colv.at[pl.ds(dst, GE)], lsem)
                pltpu.async_copy(w_h.at[pl.ds(src, GE)], wvs[par], lsem)

            def wait_a(b, par):
                src = pl.multiple_of((lo + b * G) * EC, EC)
                dst = pl.multiple_of(b * GE, EC)
                pltpu.make_async_copy(ei_h.at[1, pl.ds(src, GE)],
                                      colv.at[pl.ds(dst, GE)], lsem).wait()
                pltpu.make_async_copy(w_h.at[pl.ds(src, GE)],
                                      wvs[par], lsem).wait()

            @pl.when(nfull > 0)
            def _():
                fire_a(0, 0)

            def body_a(s, carry):
                for par in (0, 1):
                    b = 2 * s + par

                    @pl.when(b < nfull)
                    def _():
                        wait_a(b, par)

                        @pl.when(b + 1 < nfull)
                        def _():
                            fire_a(b + 1, 1 - par)
                        base = pl.multiple_of(b * GE, EC)
                        sl = pl.ds(base, GE)
                        pltpu.sync_copy(wvs[par], deg_sh.at[colv.at[sl]],
                                        add=True)
                return carry
            lax.fori_loop(0, (nfull + 1) // 2, body_a, None)

            def tail_a(c, carry):
                src = pl.multiple_of((lo + c) * EC, EC)
                dst = pl.multiple_of(c * EC, EC)
                sl = pl.ds(dst, EC)
                vs = pl.ds(0, EC)
                pltpu.sync_copy(ei_h.at[1, pl.ds(src, EC)], colv.at[sl])
                pltpu.sync_copy(w_h.at[pl.ds(src, EC)], wv0.at[vs])
                pltpu.sync_copy(wv0.at[vs], deg_sh.at[colv.at[sl]], add=True)
                return carry
            lax.fori_loop(tail_lo, nch, tail_a, None)
            plsc.subcore_barrier()

            pltpu.sync_copy(deg_sh.at[pl.ds(nb, npt)], nbuf)

            def dis_loop(j, carry):
                sl = pl.ds(j * L, L)
                nbuf[sl] = _rsqrt_newton(nbuf[sl] + 1.0)
                return carry
            lax.fori_loop(0, npt // L, dis_loop, None)
            pltpu.sync_copy(nbuf, deg_sh.at[pl.ds(nb, npt)])
            plsc.subcore_barrier()

            def fire_b(b, par):
                src = pl.multiple_of((lo + b * G) * EC, EC)
                pltpu.async_copy(ei_h.at[0, pl.ds(src, GE)], rbs[par], lsem)
                pltpu.async_copy(w_h.at[pl.ds(src, GE)], wvs[par], lsem)

            def wait_b(b, par):
                src = pl.multiple_of((lo + b * G) * EC, EC)
                pltpu.make_async_copy(ei_h.at[0, pl.ds(src, GE)],
                                      rbs[par], lsem).wait()
                pltpu.make_async_copy(w_h.at[pl.ds(src, GE)],
                                      wvs[par], lsem).wait()

            @pl.when(nfull > 0)
            def _():
                fire_b(0, 0)

            def body_b(s, carry):
                for par in (0, 1):
                    b = 2 * s + par

                    @pl.when(b < nfull)
                    def _():
                        wait_b(b, par)

                        @pl.when(b + 1 < nfull)
                        def _():
                            fire_b(b + 1, 1 - par)
                        base = pl.multiple_of(b * GE, EC)
                        sl = pl.ds(base, GE)
                        pltpu.sync_copy(deg_sh.at[colv.at[sl]], gb)
                        for q in range(GE // L):
                            vq = pl.ds(q * L, L)
                            wvs[par][vq] = wvs[par][vq] * gb[vq]
                        pltpu.sync_copy(wvs[par], u_sh.at[rbs[par]],
                                        add=True)
                return carry
            lax.fori_loop(0, (nfull + 1) // 2, body_b, None)

            def tail_b(c, carry):
                src = pl.multiple_of((lo + c) * EC, EC)
                dst = pl.multiple_of(c * EC, EC)
                sl = pl.ds(dst, EC)
                vs = pl.ds(0, EC)
                pltpu.sync_copy(ei_h.at[0, pl.ds(src, EC)], rb0.at[vs])
                pltpu.sync_copy(w_h.at[pl.ds(src, EC)], wv0.at[vs])
                pltpu.sync_copy(deg_sh.at[colv.at[sl]], gb.at[vs])
                for q in range(EC // L):
                    vq = pl.ds(q * L, L)
                    wv0[vq] = wv0[vq] * gb[vq]
                pltpu.sync_copy(wv0.at[vs], u_sh.at[rb0.at[vs]], add=True)
                return carry
            lax.fori_loop(tail_lo, nch, tail_b, None)
            plsc.subcore_barrier()

            pltpu.sync_copy(deg_sh.at[pl.ds(nb, npt)], nbuf)
            pltpu.sync_copy(u_sh.at[pl.ds(nb, npt)], wb.at[pl.ds(0, npt)])

            def c_loop(j, carry):
                sl = pl.ds(j * L, L)
                dv = nbuf[sl]
                nbuf[sl] = dv * (wb[sl] + dv)
                return carry
            lax.fori_loop(0, npt // L, c_loop, None)
            pltpu.sync_copy(nbuf, u_sh.at[pl.ds(nb, npt)])
            plsc.subcore_barrier()

            clo = t * cch_pt
            chi = jnp.minimum(clo + cch_pt, n_cch)

            def pass_c(i, carry):
                b = pl.multiple_of(i * CC, 16)
                pltpu.sync_copy(x_h.at[pl.ds(b, CC)], cxi)
                pltpu.sync_copy(u_sh.at[pl.ds(b, CC)], ccv)
                pltpu.sync_copy(ccv, cc_sh.at[cxi], add=True)
                return carry
            lax.fori_loop(clo, chi, pass_c, None)
            plsc.subcore_barrier()

            pltpu.sync_copy(cc_sh.at[pl.ds(t * spt, spt)], wb)
            pltpu.sync_copy(wb, cc_h.at[pl.ds(t * spt, spt)])

    return sc_kernel


def _sc_x1(S, B, EMB):
    NT = 16
    NW = 2 * NT
    bpt = B // NW
    assert B % NW == 0
    mesh = plsc.VectorSubcoreMesh(core_axis_name="c", subcore_axis_name="s")

    @functools.partial(
        pl.kernel,
        out_type=jax.ShapeDtypeStruct((B, EMB), jnp.float32),
        mesh=mesh,
        compiler_params=pltpu.CompilerParams(needs_layout_passes=False,
                                             use_tc_tiling_on_sc=False),
        scratch_types=[
            pltpu.VMEM((bpt,), jnp.int32),
            pltpu.VMEM((bpt, EMB), jnp.float32),
        ],
    )
    def x1_kernel(st_h, emb_h, x1_h, sti, srows):
        wid = lax.axis_index("c") * NT + lax.axis_index("s")
        b = pl.multiple_of(wid * bpt, 16)
        pltpu.sync_copy(st_h.at[pl.ds(b, bpt)], sti)
        pltpu.sync_copy(emb_h.at[sti], srows)
        pltpu.sync_copy(srows, x1_h.at[pl.ds(b, bpt)])

    return x1_kernel


def _tc_head(S, S2, B, EMB, HID, A, n_nodes):
    inv_n = 1.0 / float(n_nodes)

    def body(cc_ref, emb_ref, x1_ref, gw_ref, gb_ref, w1_ref, b1_ref,
             w2_ref, b2_ref, w3_ref, b3_ref, vw_ref, vb_ref, aw_ref, ab_ref,
             out_ref):
        cv = cc_ref[pl.ds(0, S)].reshape(1, S)
        acc = jnp.dot(cv, emb_ref[...], preferred_element_type=jnp.float32)
        s = acc * inv_n
        x2 = jnp.dot(s, gw_ref[...], preferred_element_type=jnp.float32) \
            + gb_ref[...]
        h = (jnp.dot(x1_ref[...], w1_ref[0:EMB, :],
                     preferred_element_type=jnp.float32)
             + jnp.dot(x2, w1_ref[EMB:, :],
                       preferred_element_type=jnp.float32)
             + b1_ref[...])
        h = jnp.maximum(h, 0.0)
        h = jnp.maximum(jnp.dot(h, w2_ref[...],
                                preferred_element_type=jnp.float32)
                        + b2_ref[...], 0.0)
        h = jnp.maximum(jnp.dot(h, w3_ref[...],
                                preferred_element_type=jnp.float32)
                        + b3_ref[...], 0.0)
        v = jnp.dot(h, vw_ref[...], preferred_element_type=jnp.float32) \
            + vb_ref[...]
        adv = jnp.dot(h, aw_ref[...], preferred_element_type=jnp.float32) \
            + ab_ref[...]
        out_ref[...] = v + adv - jnp.mean(adv, axis=1, keepdims=True)

    return pl.pallas_call(
        body,
        out_shape=jax.ShapeDtypeStruct((B, A), jnp.float32),
    )


def kernel(state, x, edge_index, edge_weight, emb_table, gcn_W, gcn_b,
           fc1_W, fc1_b, fc2_W, fc2_b, fc3_W, fc3_b, val_W, val_b,
           adv_W, adv_b):
    N = x.shape[0]
    E = edge_weight.shape[0]
    S, EMB = emb_table.shape
    HID = fc2_W.shape[0]
    B = state.shape[0]
    A = adv_W.shape[1]

    ei = edge_index.astype(jnp.int32)
    w32 = edge_weight.astype(jnp.float32)
    xi = x.astype(jnp.int32)
    sti = state[:, 0].astype(jnp.int32)

    cc_pad = _sc_coeffs(N, E, S, B, EMB)(ei, w32, xi)
    x1 = _sc_x1(S, B, EMB)(sti, emb_table)
    S2 = cc_pad.shape[0]

    return _tc_head(S, S2, B, EMB, HID, A, N)(
        cc_pad, emb_table, x1,
        gcn_W, gcn_b.reshape(1, HID),
        fc1_W, fc1_b.reshape(1, HID),
        fc2_W, fc2_b.reshape(1, HID),
        fc3_W, fc3_b.reshape(1, HID),
        val_W, val_b.reshape(1, 1),
        adv_W, adv_b.reshape(1, A))


# speedup vs baseline: 116.0298x; 1.4776x over previous
import functools

import jax
import jax.numpy as jnp
from jax import lax
from jax.experimental import pallas as pl
from jax.experimental.pallas import tpu as pltpu
from jax.experimental.pallas import tpu_sc as plsc

_L = 16
_NT = 16
_EC = 128
_G = 16
_GE = _G * _EC

_SC_PARAMS = pltpu.CompilerParams(needs_layout_passes=False,
                                  use_tc_tiling_on_sc=True)
_MESH = dict(core_axis_name="c", subcore_axis_name="s")


def _rsqrt_newton(d):
    i = plsc.bitcast(d, jnp.int32)
    i = jnp.int32(0x5F3759DF) - lax.shift_right_logical(i, 1)
    y = plsc.bitcast(i, jnp.float32)
    for _ in range(3):
        y = y * (1.5 - 0.5 * d * y * y)
    return y


def _zero_spmem(buf, shared, base, n):
    def zl(j, carry):
        buf[pl.ds(j * _L, _L)] = jnp.zeros((_L,), jnp.float32)
        return carry
    lax.fori_loop(0, n // _L, zl, None)
    pltpu.sync_copy(buf.at[pl.ds(0, n)], shared.at[pl.ds(base, n)])


def _sc_deg(N2, E):
    npt = N2 // _NT
    n_ech = E // _EC
    half = n_ech // 2
    ech_pt = -(-half // _NT)

    @functools.partial(
        pl.kernel,
        out_type=jax.ShapeDtypeStruct((2 * N2,), jnp.float32),
        mesh=plsc.VectorSubcoreMesh(**_MESH),
        compiler_params=_SC_PARAMS,
        scratch_types=[
            pltpu.VMEM_SHARED((N2,), jnp.float32),
            pltpu.VMEM((npt,), jnp.float32),
            pltpu.VMEM((_GE,), jnp.int32),
            pltpu.VMEM((_GE,), jnp.int32),
            pltpu.VMEM((_GE,), jnp.float32),
            pltpu.VMEM((_GE,), jnp.float32),
            pltpu.SemaphoreType.DMA,
        ],
    )
    def k(ei_h, w_h, degp_h, deg_sh, nbuf, cb0, cb1, wv0, wv1, lsem):
        cid = lax.axis_index("c")
        t = lax.axis_index("s")
        cbs, wvs = (cb0, cb1), (wv0, wv1)
        nb = t * npt
        _zero_spmem(nbuf, deg_sh, nb, npt)
        plsc.subcore_barrier()

        lo = cid * half + t * ech_pt
        hi = jnp.minimum(lo + ech_pt, cid * half + half)
        nfull = (hi - lo) // _G
        tail_lo = nfull * _G

        def fire(b, par):
            src = pl.multiple_of((lo + b * _G) * _EC, _EC)
            pltpu.async_copy(ei_h.at[1, pl.ds(src, _GE)], cbs[par], lsem)
            pltpu.async_copy(w_h.at[pl.ds(src, _GE)], wvs[par], lsem)

        def wait(b, par):
            src = pl.multiple_of((lo + b * _G) * _EC, _EC)
            pltpu.make_async_copy(ei_h.at[1, pl.ds(src, _GE)],
                                  cbs[par], lsem).wait()
            pltpu.make_async_copy(w_h.at[pl.ds(src, _GE)],
                                  wvs[par], lsem).wait()

        @pl.when(nfull > 0)
        def _():
            fire(0, 0)

        def body(s, carry):
            for par in (0, 1):
                b = 2 * s + par

                @pl.when(b < nfull)
                def _():
                    wait(b, par)

                    @pl.when(b + 1 < nfull)
                    def _():
                        fire(b + 1, 1 - par)
                    pltpu.sync_copy(wvs[par], deg_sh.at[cbs[par]], add=True)
            return carry
        lax.fori_loop(0, (nfull + 1) // 2, body, None)

        def tail(c, carry):
            src = pl.multiple_of((lo + c) * _EC, _EC)
            sl = pl.ds(0, _EC)
            pltpu.sync_copy(ei_h.at[1, pl.ds(src, _EC)], cb0.at[sl])
            pltpu.sync_copy(w_h.at[pl.ds(src, _EC)], wv0.at[sl])
            pltpu.sync_copy(wv0.at[sl], deg_sh.at[cb0.at[sl]], add=True)
            return carry
        lax.fori_loop(tail_lo, hi - lo, tail, None)
        plsc.subcore_barrier()

        pltpu.sync_copy(deg_sh.at[pl.ds(nb, npt)], nbuf)
        pltpu.sync_copy(nbuf, degp_h.at[pl.ds(cid * N2 + nb, npt)])

    return k


def _sc_u(N2, E):
    npt = N2 // _NT
    n_ech = E // _EC
    half = n_ech // 2
    ech_pt = -(-half // _NT)

    @functools.partial(
        pl.kernel,
        out_type=jax.ShapeDtypeStruct((2 * N2,), jnp.float32),
        mesh=plsc.VectorSubcoreMesh(**_MESH),
        compiler_params=_SC_PARAMS,
        scratch_types=[
            pltpu.VMEM_SHARED((N2,), jnp.float32),
            pltpu.VMEM_SHARED((N2,), jnp.float32),
            pltpu.VMEM((npt,), jnp.float32),
            pltpu.VMEM((npt,), jnp.float32),
            pltpu.VMEM((_GE,), jnp.int32),
            pltpu.VMEM((_GE,), jnp.int32),
            pltpu.VMEM((_GE,), jnp.int32),
            pltpu.VMEM((_GE,), jnp.int32),
            pltpu.VMEM((_GE,), jnp.float32),
            pltpu.VMEM((_GE,), jnp.float32),
            pltpu.VMEM((_GE,), jnp.float32),
            pltpu.SemaphoreType.DMA,
        ],
    )
    def k(ei_h, w_h, degp_h, up_h, dis_sh, u_sh, na, nbb,
          cb0, cb1, rb0, rb1, wv0, wv1, gb, lsem):
        cid = lax.axis_index("c")
        t = lax.axis_index("s")
        cbs, rbs, wvs = (cb0, cb1), (rb0, rb1), (wv0, wv1)
        nb = t * npt

        pltpu.sync_copy(degp_h.at[pl.ds(nb, npt)], na)
        pltpu.sync_copy(degp_h.at[pl.ds(N2 + nb, npt)], nbb)

        def dl(j, carry):
            sl = pl.ds(j * _L, _L)
            na[sl] = _rsqrt_newton(na[sl] + nbb[sl] + 1.0)
            return carry
        lax.fori_loop(0, npt // _L, dl, None)
        pltpu.sync_copy(na, dis_sh.at[pl.ds(nb, npt)])
        _zero_spmem(nbb, u_sh, nb, npt)
        plsc.subcore_barrier()

        lo = cid * half + t * ech_pt
        hi = jnp.minimum(lo + ech_pt, cid * half + half)
        nfull = (hi - lo) // _G
        tail_lo = nfull * _G

        def fire(b, par):
            src = pl.multiple_of((lo + b * _G) * _EC, _EC)
            pltpu.async_copy(ei_h.at[1, pl.ds(src, _GE)], cbs[par], lsem)
            pltpu.async_copy(ei_h.at[0, pl.ds(src, _GE)], rbs[par], lsem)
            pltpu.async_copy(w_h.at[pl.ds(src, _GE)], wvs[par], lsem)

        def wait(b, par):
            src = pl.multiple_of((lo + b * _G) * _EC, _EC)
            pltpu.make_async_copy(ei_h.at[1, pl.ds(src, _GE)],
                                  cbs[par], lsem).wait()
            pltpu.make_async_copy(ei_h.at[0, pl.ds(src, _GE)],
                                  rbs[par], lsem).wait()
            pltpu.make_async_copy(w_h.at[pl.ds(src, _GE)],
                                  wvs[par], lsem).wait()

        @pl.when(nfull > 0)
        def _():
            fire(0, 0)

        def body(s, carry):
            for par in (0, 1):
                b = 2 * s + par

                @pl.when(b < nfull)
                def _():
                    wait(b, par)

                    @pl.when(b + 1 < nfull)
                    def _():
                        fire(b + 1, 1 - par)
                    pltpu.sync_copy(dis_sh.at[cbs[par]], gb)
                    for q in range(_GE // _L):
                        vq = pl.ds(q * _L, _L)
                        wvs[par][vq] = wvs[par][vq] * gb[vq]
                    pltpu.sync_copy(wvs[par], u_sh.at[rbs[par]], add=True)
            return carry
        lax.fori_loop(0, (nfull + 1) // 2, body, None)

        def tail(c, carry):
            src = pl.multiple_of((lo + c) * _EC, _EC)
            sl = pl.ds(0, _EC)
            pltpu.sync_copy(ei_h.at[1, pl.ds(src, _EC)], cb0.at[sl])
            pltpu.sync_copy(ei_h.at[0, pl.ds(src, _EC)], rb0.at[sl])
            pltpu.sync_copy(w_h.at[pl.ds(src, _EC)], wv0.at[sl])
            pltpu.sync_copy(dis_sh.at[cb0.at[sl]], gb.at[sl])
            for q in range(_EC // _L):
                vq = pl.ds(q * _L, _L)
                wv0[vq] = wv0[vq] * gb[vq]
            pltpu.sync_copy(wv0.at[sl], u_sh.at[rb0.at[sl]], add=True)
            return carry
        lax.fori_loop(tail_lo, hi - lo, tail, None)
        plsc.subcore_barrier()

        pltpu.sync_copy(u_sh.at[pl.ds(nb, npt)], na)
        pltpu.sync_copy(na, up_h.at[pl.ds(cid * N2 + nb, npt)])

    return k


def _sc_cc(N, N2, S2):
    npt = N2 // _NT
    spt = S2 // _NT
    CC = 80
    assert N % CC == 0
    n_cch = N // CC
    halfc = -(-n_cch // 2)
    cch_pt = -(-halfc // _NT)

    @functools.partial(
        pl.kernel,
        out_type=jax.ShapeDtypeStruct((2 * S2,), jnp.float32),
        mesh=plsc.VectorSubcoreMesh(**_MESH),
        compiler_params=_SC_PARAMS,
        scratch_types=[
            pltpu.VMEM_SHARED((N2,), jnp.float32),
            pltpu.VMEM_SHARED((S2,), jnp.float32),
            pltpu.VMEM((spt,), jnp.float32),
            pltpu.VMEM((npt,), jnp.float32),
            pltpu.VMEM((npt,), jnp.float32),
            pltpu.VMEM((CC,), jnp.int32),
            pltpu.VMEM((CC,), jnp.float32),
        ],
    )
    def k(x_h, degp_h, up_h, ccp_h, c_sh, cc_sh, wb, na, nbb, cxi, ccv):
        cid = lax.axis_index("c")
        t = lax.axis_index("s")
        nb = t * npt

        pltpu.sync_copy(degp_h.at[pl.ds(nb, npt)], na)
        pltpu.sync_copy(degp_h.at[pl.ds(N2 + nb, npt)], nbb)

        def dl(j, carry):
            sl = pl.ds(j * _L, _L)
            na[sl] = _rsqrt_newton(na[sl] + nbb[sl] + 1.0)
            return carry
        lax.fori_loop(0, npt // _L, dl, None)
        pltpu.sync_copy(up_h.at[pl.ds(nb, npt)], nbb)
        pltpu.sync_copy(up_h.at[pl.ds(N2 + nb, npt)],
                        wb.at[pl.ds(0, npt)])

        def cl(j, carry):
            sl = pl.ds(j * _L, _L)
            dv = na[sl]
            na[sl] = dv * (nbb[sl] + wb[sl] + dv)
            return carry
        lax.fori_loop(0, npt // _L, cl, None)
        pltpu.sync_copy(na, c_sh.at[pl.ds(nb, npt)])
        _zero_spmem(wb, cc_sh, t * spt, spt)
        plsc.subcore_barrier()

        clo = cid * halfc + t * cch_pt
        chi = jnp.minimum(clo + cch_pt,
                          jnp.minimum(cid * halfc + halfc, n_cch))

        def pc(i, carry):
            b = pl.multiple_of(i * CC, 16)
            pltpu.sync_copy(x_h.at[pl.ds(b, CC)], cxi)
            pltpu.sync_copy(c_sh.at[pl.ds(b, CC)], ccv)
            pltpu.sync_copy(ccv, cc_sh.at[cxi], add=True)
            return carry
        lax.fori_loop(clo, chi, pc, None)
        plsc.subcore_barrier()

        pltpu.sync_copy(cc_sh.at[pl.ds(t * spt, spt)], wb)
        pltpu.sync_copy(wb, ccp_h.at[pl.ds(cid * S2 + t * spt, spt)])

    return k


def _sc_x1(B, EMB):
    NW = 2 * _NT
    bpt = B // NW
    assert B % NW == 0

    @functools.partial(
        pl.kernel,
        out_type=jax.ShapeDtypeStruct((B, EMB), jnp.float32),
        mesh=plsc.VectorSubcoreMesh(**_MESH),
        compiler_params=pltpu.CompilerParams(needs_layout_passes=False,
                                             use_tc_tiling_on_sc=False),
        scratch_types=[
            pltpu.VMEM((bpt,), jnp.int32),
            pltpu.VMEM((bpt, EMB), jnp.float32),
        ],
    )
    def k(st_h, emb_h, x1_h, sti, srows):
        wid = lax.axis_index("c") * _NT + lax.axis_index("s")
        b = pl.multiple_of(wid * bpt, 16)
        pltpu.sync_copy(st_h.at[pl.ds(b, bpt)], sti)
        pltpu.sync_copy(emb_h.at[sti], srows)
        pltpu.sync_copy(srows, x1_h.at[pl.ds(b, bpt)])

    return k


def _tc_head(S, S2, B, EMB, HID, A, n_nodes):
    inv_n = 1.0 / float(n_nodes)

    def body(cc_ref, emb_ref, x1_ref, gw_ref, gb_ref, w1_ref, b1_ref,
             w2_ref, b2_ref, w3_ref, b3_ref, vw_ref, vb_ref, aw_ref, ab_ref,
             out_ref):
        cv = (cc_ref[pl.ds(0, S)] + cc_ref[pl.ds(S2, S)]).reshape(1, S)
        acc = jnp.dot(cv, emb_ref[...], preferred_element_type=jnp.float32)
        s = acc * inv_n
        x2 = jnp.dot(s, gw_ref[...], preferred_element_type=jnp.float32) \
            + gb_ref[...]
        h = (jnp.dot(x1_ref[...], w1_ref[0:EMB, :],
                     preferred_element_type=jnp.float32)
             + jnp.dot(x2, w1_ref[EMB:, :],
                       preferred_element_type=jnp.float32)
             + b1_ref[...])
        h = jnp.maximum(h, 0.0)
        h = jnp.maximum(jnp.dot(h, w2_ref[...],
                                preferred_element_type=jnp.float32)
                        + b2_ref[...], 0.0)
        h = jnp.maximum(jnp.dot(h, w3_ref[...],
                                preferred_element_type=jnp.float32)
                        + b3_ref[...], 0.0)
        v = jnp.dot(h, vw_ref[...], preferred_element_type=jnp.float32) \
            + vb_ref[...]
        adv = jnp.dot(h, aw_ref[...], preferred_element_type=jnp.float32) \
            + ab_ref[...]
        out_ref[...] = v + adv - jnp.mean(adv, axis=1, keepdims=True)

    return pl.pallas_call(
        body,
        out_shape=jax.ShapeDtypeStruct((B, A), jnp.float32),
    )


def kernel(state, x, edge_index, edge_weight, emb_table, gcn_W, gcn_b,
           fc1_W, fc1_b, fc2_W, fc2_b, fc3_W, fc3_b, val_W, val_b,
           adv_W, adv_b):
    N = x.shape[0]
    E = edge_weight.shape[0]
    S, EMB = emb_table.shape
    HID = fc2_W.shape[0]
    B = state.shape[0]
    A = adv_W.shape[1]
    NL = _NT * _L
    N2 = ((N + NL - 1) // NL) * NL
    S2 = ((S + NL - 1) // NL) * NL

    ei = edge_index.astype(jnp.int32)
    w32 = edge_weight.astype(jnp.float32)
    xi = x.astype(jnp.int32)
    sti = state[:, 0].astype(jnp.int32)

    degp = _sc_deg(N2, E)(ei, w32)
    up = _sc_u(N2, E)(ei, w32, degp)
    ccp = _sc_cc(N, N2, S2)(xi, degp, up)
    x1 = _sc_x1(B, EMB)(sti, emb_table)

    return _tc_head(S, S2, B, EMB, HID, A, N)(
        ccp, emb_table, x1,
        gcn_W, gcn_b.reshape(1, HID),
        fc1_W, fc1_b.reshape(1, HID),
        fc2_W, fc2_b.reshape(1, HID),
        fc3_W, fc3_b.reshape(1, HID),
        val_W, val_b.reshape(1, 1),
        adv_W, adv_b.reshape(1, A))
